# baseline (speedup 1.0000x reference)
import functools
import jax
import jax.numpy as jnp
from jax import lax
from jax.experimental import pallas as pl
from jax.experimental.pallas import tpu as pltpu, tpu_sc as plsc

N = 10000
D = 128
H = 128
E = 320000

NS = 16
CHUNK = 64
RING = 5
SRING = 2
STAGES = 320
UNROLL = 10
EP_TILE = STAGES * CHUNK
EP = EP_TILE * NS
WB_TILES = 10
ROWS_PER_TILE = N // WB_TILES

_PERM = [32 * b + (j // 2 if j % 2 == 0 else 16 + j // 2)
         for b in range(H // 32) for j in range(32)]


def _pre_body(h_ref, w_ref, out_ref):
    out_ref[...] = jnp.tanh(
        jax.lax.dot_general(h_ref[...], w_ref[0],
                            (((1,), (1,)), ((), ())),
                            preferred_element_type=jnp.float32)
    ).astype(jnp.bfloat16)


def _make_table(h, W0, W1):
    blk = 2000
    nblk = N // blk
    perm = jnp.asarray(_PERM, dtype=jnp.int32)
    Ws = jnp.stack([W0[perm], W1[perm]])
    tab = pl.pallas_call(
        _pre_body,
        grid=(2, nblk),
        in_specs=[
            pl.BlockSpec((blk, D), lambda m, i: (i, 0)),
            pl.BlockSpec((1, H, D), lambda m, i: (m, 0, 0)),
        ],
        out_specs=pl.BlockSpec((blk, H), lambda m, i: (m * nblk + i, 0)),
        out_shape=jax.ShapeDtypeStruct((2 * N, H), jnp.bfloat16),
    )(h, Ws)
    return lax.bitcast_convert_type(tab.reshape(2 * N, H // 2, 2), jnp.int32)


def _sc_body(table, packed, ews, binit, e0, e1,
             meta_v, ew_v, msgi_v, msgf_v, acc, *sems):
    sg = sems[0:RING]
    ss = sems[RING:RING + SRING]
    sm = sems[RING + SRING:2 * RING + SRING]
    c = lax.axis_index("c")
    s = lax.axis_index("s")

    @pl.when(s < WB_TILES)
    def _():
        pltpu.sync_copy(binit.at[c],
                        acc.at[pl.ds(s * ROWS_PER_TILE, ROWS_PER_TILE)])
    plsc.subcore_barrier()

    dnums = lax.GatherDimensionNumbers(
        offset_dims=(), collapsed_slice_dims=(0,), start_index_map=(0,))

    def fire_meta(stage, u):
        pltpu.async_copy(packed.at[c, s, stage], meta_v.at[u], sm[u])
        pltpu.async_copy(ews.at[c, s, stage], ew_v.at[u], sm[u])

    def wait_meta(u):
        pltpu.make_async_copy(packed.at[c, s, 0], meta_v.at[u], sm[u]).wait()
        pltpu.make_async_copy(ews.at[c, s, 0], ew_v.at[u], sm[u]).wait()

    def fire_gather(u):
        pltpu.async_copy(table.at[meta_v.at[u, 0]], msgi_v.at[u], sg[u])

    def wait_gather(u):
        pltpu.make_async_copy(table.at[meta_v.at[u, 0]], msgi_v.at[u],
                              sg[u]).wait()

    def fire_scatter(u, b):
        pltpu.async_copy(msgf_v.at[b], acc.at[meta_v.at[u, 1]], ss[b],
                         add=True)

    def wait_scatter(u, b):
        pltpu.make_async_copy(msgf_v.at[b], acc.at[meta_v.at[u, 1]],
                              ss[b]).wait()

    for u in range(4):
        fire_meta(u, u)
    for u in range(3):
        wait_meta(u)
        fire_gather(u)

    def outer_body(i, _):
        for t in range(UNROLL):
            k = i * UNROLL + t
            u = t % RING
            b = t % SRING

            @pl.when(k > 0)
            def _():
                wait_scatter((u + RING - 1) % RING, (b + 1) % SRING)

            @pl.when(k + 4 < STAGES)
            def _():
                fire_meta(k + 4, (u + 4) % RING)

            wait_gather(u)

            def grp_body(g, _):
                ew16 = ew_v[u, pl.ds(g * 16, 16)]
                for j in range(16):
                    w = lax.gather(ew16, jnp.full((16, 1), j, jnp.int32),
                                   dnums, (1,),
                                   mode=lax.GatherScatterMode.PROMISE_IN_BOUNDS)
                    e = g * 16 + j
                    for q in range(H // 32):
                        v = msgi_v[u, e, pl.ds(q * 16, 16)]
                        lo = plsc.bitcast(lax.shift_left(v, 16), jnp.float32)
                        hi = plsc.bitcast(
                            lax.bitwise_and(v, jnp.int32(-65536)), jnp.float32)
                        msgf_v[b, e, pl.ds(q * 32, 16)] = lo * w
                        msgf_v[b, e, pl.ds(q * 32 + 16, 16)] = hi * w
                return 0

            lax.fori_loop(0, CHUNK // 16, grp_body, 0)
            fire_scatter(u, b)

            @pl.when(k + 3 < STAGES)
            def _():
                wait_meta((u + 3) % RING)
                fire_gather((u + 3) % RING)
        return 0

    lax.fori_loop(0, STAGES // UNROLL, outer_body, 0)
    wait_scatter((STAGES - 1) % RING, (STAGES - 1) % SRING)
    plsc.subcore_barrier()

    sl = pl.ds(s * ROWS_PER_TILE, ROWS_PER_TILE)

    @pl.when(jnp.logical_and(c == 0, s < WB_TILES))
    def _():
        pltpu.sync_copy(acc.at[sl], e0.at[sl])

    @pl.when(jnp.logical_and(c == 1, s < WB_TILES))
    def _():
        pltpu.sync_copy(acc.at[sl], e1.at[sl])


def _segment_spmm(table, packed, ews, binit):
    mesh = plsc.VectorSubcoreMesh(core_axis_name="c", subcore_axis_name="s")
    f = functools.partial(
        pl.kernel,
        out_type=(jax.ShapeDtypeStruct((N, H), jnp.float32),
                  jax.ShapeDtypeStruct((N, H), jnp.float32)),
        mesh=mesh,
        compiler_params=pltpu.CompilerParams(needs_layout_passes=False, use_tc_tiling_on_sc=False),
        scratch_types=[
            pltpu.VMEM((RING, 2, CHUNK), jnp.int32),
            pltpu.VMEM((RING, CHUNK), jnp.float32),
            pltpu.VMEM((RING, CHUNK, H // 2), jnp.int32),
            pltpu.VMEM((SRING, CHUNK, H), jnp.float32),
            pltpu.VMEM_SHARED((N, H), jnp.float32),
        ] + [pltpu.SemaphoreType.DMA] * (2 * RING + SRING),
    )(_sc_body)
    return f(table, packed, ews, binit)


def _sums_body(e0_ref, e1_ref, fcw_ref, fcb_ref, out_ref):
    i = pl.program_id(0)
    dn = (((1,), (1,)), ((), ()))
    t0 = jnp.tanh(jax.lax.dot_general(e0_ref[...], fcw_ref[...], dn,
                                      preferred_element_type=jnp.float32)
                  + fcb_ref[...])
    t1 = jnp.tanh(jax.lax.dot_general(e1_ref[...], fcw_ref[...], dn,
                                      preferred_element_type=jnp.float32)
                  + fcb_ref[...])
    part = jnp.stack([jnp.sum(t0, axis=0), jnp.sum(t1, axis=0)])

    @pl.when(i == 0)
    def _():
        out_ref[...] = part

    @pl.when(i > 0)
    def _():
        out_ref[...] = out_ref[...] + part


def _attn_sums(e0, e1, fc_W, fc_b):
    blk = 1000
    nblk = N // blk
    return pl.pallas_call(
        _sums_body,
        grid=(nblk,),
        in_specs=[
            pl.BlockSpec((blk, H), lambda i: (i, 0)),
            pl.BlockSpec((blk, H), lambda i: (i, 0)),
            pl.BlockSpec((H, H), lambda i: (0, 0)),
            pl.BlockSpec((1, H), lambda i: (0, 0)),
        ],
        out_specs=pl.BlockSpec((2, H), lambda i: (0, 0)),
        out_shape=jax.ShapeDtypeStruct((2, H), jnp.float32),
    )(e0, e1, fc_W, fc_b.reshape(1, H))


def _combine_body(sums_ref, att_ref, e0_ref, e1_ref, z_ref):
    sp = sums_ref[...] * (1.0 / N)
    logits = jnp.sum(att_ref[...] * sp, axis=1)
    m = jnp.max(logits)
    ex = jnp.exp(logits - m)
    beta = ex / jnp.sum(ex)
    z_ref[...] = beta[0] * e0_ref[...] + beta[1] * e1_ref[...]


def _combine(sums, att, e0, e1):
    blk = 1000
    nblk = N // blk
    return pl.pallas_call(
        _combine_body,
        grid=(nblk,),
        in_specs=[
            pl.BlockSpec((2, H), lambda i: (0, 0)),
            pl.BlockSpec((1, H), lambda i: (0, 0)),
            pl.BlockSpec((blk, H), lambda i: (i, 0)),
            pl.BlockSpec((blk, H), lambda i: (i, 0)),
        ],
        out_specs=pl.BlockSpec((blk, H), lambda i: (i, 0)),
        out_shape=jax.ShapeDtypeStruct((N, H), jnp.float32),
    )(sums, att, e0, e1)


def _pack_edges(col, row, ew):
    pad = EP - E
    col = jnp.concatenate([col, jnp.zeros((pad,), col.dtype)])
    row = jnp.concatenate([row, jnp.zeros((pad,), row.dtype)])
    ew = jnp.concatenate([ew, jnp.zeros((pad,), ew.dtype)])
    idx = jnp.stack([col.reshape(NS, STAGES, CHUNK),
                     row.reshape(NS, STAGES, CHUNK)], axis=2)
    return idx, ew.reshape(NS, STAGES, CHUNK)


def kernel(h, mps_edge_index_0, mps_edge_weight_0,
           mps_edge_index_1, mps_edge_weight_1,
           W0, b0, W1, b1, fc_W, fc_b, att):
    col0 = mps_edge_index_0[1].astype(jnp.int32)
    row0 = mps_edge_index_0[0].astype(jnp.int32)
    col1 = mps_edge_index_1[1].astype(jnp.int32) + N
    row1 = mps_edge_index_1[0].astype(jnp.int32)
    idx0, ewr0 = _pack_edges(col0, row0, mps_edge_weight_0)
    idx1, ewr1 = _pack_edges(col1, row1, mps_edge_weight_1)
    packed = jnp.stack([idx0, idx1])
    ews = jnp.stack([ewr0, ewr1])
    binit = jnp.stack([
        jnp.broadcast_to(b0[None, :], (ROWS_PER_TILE, H)),
        jnp.broadcast_to(b1[None, :], (ROWS_PER_TILE, H)),
    ])

    table = _make_table(h, W0, W1)
    e0, e1 = _segment_spmm(table, packed, ews, binit)
    sums = _attn_sums(e0, e1, fc_W, fc_b)
    z = _combine(sums, att, e0, e1)
    return (z, e0, e1)


# speedup vs baseline: 1.7127x; 1.7127x over previous
import functools
import jax
import jax.numpy as jnp
from jax import lax
from jax.experimental import pallas as pl
from jax.experimental.pallas import tpu as pltpu, tpu_sc as plsc

N = 10000
D = 128
H = 128
E = 320000

NS = 16
CHUNK = 64
RING = 5
STAGES = 315
UNROLL = 5
EP_TILE = STAGES * CHUNK
EP = EP_TILE * NS
WB_TILES = 10
ROWS_PER_TILE = N // WB_TILES

_PERM = [32 * b + (j // 2 if j % 2 == 0 else 16 + j // 2)
         for b in range(H // 32) for j in range(32)]


def _pre_body(h_ref, w_ref, out_ref):
    out_ref[...] = jnp.tanh(
        jax.lax.dot_general(h_ref[...], w_ref[0],
                            (((1,), (1,)), ((), ())),
                            preferred_element_type=jnp.float32))


def _make_table(h, W0, W1):
    blk = 2000
    nblk = N // blk
    Ws = jnp.stack([W0, W1])
    tab = pl.pallas_call(
        _pre_body,
        grid=(2, nblk),
        in_specs=[
            pl.BlockSpec((blk, D), lambda m, i: (i, 0)),
            pl.BlockSpec((1, H, D), lambda m, i: (m, 0, 0)),
        ],
        out_specs=pl.BlockSpec((blk, H), lambda m, i: (m * nblk + i, 0)),
        out_shape=jax.ShapeDtypeStruct((2 * N, H), jnp.float32),
    )(h, Ws)
    return tab


def _sc_body(table, packed, ews, binit, e0, e1,
             meta_v, ew_v, msg_v, acc, *sems):
    sg = sems[0:RING]
    ss = sems[RING:2 * RING]
    sm = sems[2 * RING:3 * RING]
    c = lax.axis_index("c")
    s = lax.axis_index("s")

    @pl.when(s < WB_TILES)
    def _():
        pltpu.sync_copy(binit.at[c],
                        acc.at[pl.ds(s * ROWS_PER_TILE, ROWS_PER_TILE)])
    plsc.subcore_barrier()

    dnums = lax.GatherDimensionNumbers(
        offset_dims=(), collapsed_slice_dims=(0,), start_index_map=(0,))

    def fire_meta(stage, u):
        pltpu.async_copy(packed.at[c, s, stage], meta_v.at[u], sm[u])
        pltpu.async_copy(ews.at[c, s, stage], ew_v.at[u], sm[u])

    def wait_meta(u):
        pltpu.make_async_copy(packed.at[c, s, 0], meta_v.at[u], sm[u]).wait()
        pltpu.make_async_copy(ews.at[c, s, 0], ew_v.at[u], sm[u]).wait()

    def fire_gather(u):
        pltpu.async_copy(table.at[meta_v.at[u, 0]], msg_v.at[u], sg[u])

    def wait_gather(u):
        pltpu.make_async_copy(table.at[meta_v.at[u, 0]], msg_v.at[u],
                              sg[u]).wait()

    def fire_scatter(u):
        pltpu.async_copy(msg_v.at[u], acc.at[meta_v.at[u, 1]], ss[u],
                         add=True)

    def wait_scatter(u):
        pltpu.make_async_copy(msg_v.at[u], acc.at[meta_v.at[u, 1]],
                              ss[u]).wait()

    for u in range(4):
        fire_meta(u, u)
    for u in range(3):
        wait_meta(u)
        fire_gather(u)

    def outer_body(i, _):
        for t in range(UNROLL):
            k = i * UNROLL + t
            u = t % RING

            @pl.when(k > 0)
            def _():
                wait_scatter((u + RING - 1) % RING)

            @pl.when(k + 4 < STAGES)
            def _():
                fire_meta(k + 4, (u + 4) % RING)

            wait_gather(u)

            def grp_body(g, _):
                ew16 = ew_v[u, pl.ds(g * 16, 16)]
                for j in range(16):
                    w = lax.gather(ew16, jnp.full((16, 1), j, jnp.int32),
                                   dnums, (1,),
                                   mode=lax.GatherScatterMode.PROMISE_IN_BOUNDS)
                    e = g * 16 + j
                    for f in range(H // 16):
                        sl = pl.ds(f * 16, 16)
                        msg_v[u, e, sl] = msg_v[u, e, sl] * w
                return 0

            lax.fori_loop(0, CHUNK // 16, grp_body, 0)
            fire_scatter(u)

            @pl.when(k + 3 < STAGES)
            def _():
                wait_meta((u + 3) % RING)
                fire_gather((u + 3) % RING)
        return 0

    lax.fori_loop(0, STAGES // UNROLL, outer_body, 0)
    wait_scatter((STAGES - 1) % RING)
    plsc.subcore_barrier()

    sl = pl.ds(s * ROWS_PER_TILE, ROWS_PER_TILE)

    @pl.when(jnp.logical_and(c == 0, s < WB_TILES))
    def _():
        pltpu.sync_copy(acc.at[sl], e0.at[sl])

    @pl.when(jnp.logical_and(c == 1, s < WB_TILES))
    def _():
        pltpu.sync_copy(acc.at[sl], e1.at[sl])


def _segment_spmm(table, packed, ews, binit):
    mesh = plsc.VectorSubcoreMesh(core_axis_name="c", subcore_axis_name="s")
    f = functools.partial(
        pl.kernel,
        out_type=(jax.ShapeDtypeStruct((N, H), jnp.float32),
                  jax.ShapeDtypeStruct((N, H), jnp.float32)),
        mesh=mesh,
        scratch_types=[
            pltpu.VMEM((RING, 2, CHUNK), jnp.int32),
            pltpu.VMEM((RING, CHUNK), jnp.float32),
            pltpu.VMEM((RING, CHUNK, H), jnp.float32),
            pltpu.VMEM_SHARED((N, H), jnp.float32),
        ] + [pltpu.SemaphoreType.DMA] * (3 * RING),
    )(_sc_body)
    return f(table, packed, ews, binit)


def _sums_body(e0_ref, e1_ref, fcw_ref, fcb_ref, out_ref):
    i = pl.program_id(0)
    dn = (((1,), (1,)), ((), ()))
    t0 = jnp.tanh(jax.lax.dot_general(e0_ref[...], fcw_ref[...], dn,
                                      preferred_element_type=jnp.float32)
                  + fcb_ref[...])
    t1 = jnp.tanh(jax.lax.dot_general(e1_ref[...], fcw_ref[...], dn,
                                      preferred_element_type=jnp.float32)
                  + fcb_ref[...])
    part = jnp.stack([jnp.sum(t0, axis=0), jnp.sum(t1, axis=0)])

    @pl.when(i == 0)
    def _():
        out_ref[...] = part

    @pl.when(i > 0)
    def _():
        out_ref[...] = out_ref[...] + part


def _attn_sums(e0, e1, fc_W, fc_b):
    blk = 1000
    nblk = N // blk
    return pl.pallas_call(
        _sums_body,
        grid=(nblk,),
        in_specs=[
            pl.BlockSpec((blk, H), lambda i: (i, 0)),
            pl.BlockSpec((blk, H), lambda i: (i, 0)),
            pl.BlockSpec((H, H), lambda i: (0, 0)),
            pl.BlockSpec((1, H), lambda i: (0, 0)),
        ],
        out_specs=pl.BlockSpec((2, H), lambda i: (0, 0)),
        out_shape=jax.ShapeDtypeStruct((2, H), jnp.float32),
    )(e0, e1, fc_W, fc_b.reshape(1, H))


def _combine_body(sums_ref, att_ref, e0_ref, e1_ref, z_ref):
    sp = sums_ref[...] * (1.0 / N)
    logits = jnp.sum(att_ref[...] * sp, axis=1)
    m = jnp.max(logits)
    ex = jnp.exp(logits - m)
    beta = ex / jnp.sum(ex)
    z_ref[...] = beta[0] * e0_ref[...] + beta[1] * e1_ref[...]


def _combine(sums, att, e0, e1):
    blk = 1000
    nblk = N // blk
    return pl.pallas_call(
        _combine_body,
        grid=(nblk,),
        in_specs=[
            pl.BlockSpec((2, H), lambda i: (0, 0)),
            pl.BlockSpec((1, H), lambda i: (0, 0)),
            pl.BlockSpec((blk, H), lambda i: (i, 0)),
            pl.BlockSpec((blk, H), lambda i: (i, 0)),
        ],
        out_specs=pl.BlockSpec((blk, H), lambda i: (i, 0)),
        out_shape=jax.ShapeDtypeStruct((N, H), jnp.float32),
    )(sums, att, e0, e1)


def _pack_edges(col, row, ew):
    pad = EP - E
    col = jnp.concatenate([col, jnp.zeros((pad,), col.dtype)])
    row = jnp.concatenate([row, jnp.zeros((pad,), row.dtype)])
    ew = jnp.concatenate([ew, jnp.zeros((pad,), ew.dtype)])
    idx = jnp.stack([col.reshape(NS, STAGES, CHUNK),
                     row.reshape(NS, STAGES, CHUNK)], axis=2)
    return idx, ew.reshape(NS, STAGES, CHUNK)


def kernel(h, mps_edge_index_0, mps_edge_weight_0,
           mps_edge_index_1, mps_edge_weight_1,
           W0, b0, W1, b1, fc_W, fc_b, att):
    col0 = mps_edge_index_0[1].astype(jnp.int32)
    row0 = mps_edge_index_0[0].astype(jnp.int32)
    col1 = mps_edge_index_1[1].astype(jnp.int32) + N
    row1 = mps_edge_index_1[0].astype(jnp.int32)
    idx0, ewr0 = _pack_edges(col0, row0, mps_edge_weight_0)
    idx1, ewr1 = _pack_edges(col1, row1, mps_edge_weight_1)
    packed = jnp.stack([idx0, idx1])
    ews = jnp.stack([ewr0, ewr1])
    binit = jnp.stack([
        jnp.broadcast_to(b0[None, :], (ROWS_PER_TILE, H)),
        jnp.broadcast_to(b1[None, :], (ROWS_PER_TILE, H)),
    ])

    table = _make_table(h, W0, W1)
    e0, e1 = _segment_spmm(table, packed, ews, binit)
    sums = _attn_sums(e0, e1, fc_W, fc_b)
    z = _combine(sums, att, e0, e1)
    return (z, e0, e1)


# speedup vs baseline: 3.2509x; 1.8981x over previous
import functools
import jax
import jax.numpy as jnp
from jax import lax
from jax.experimental import pallas as pl
from jax.experimental.pallas import tpu as pltpu, tpu_sc as plsc

N = 10000
D = 128
H = 128
E = 320000

NS = 16
E_TILE = E // NS
CHUNK = 64
RING = 5
UNROLL = 5
STAGES = E_TILE // CHUNK
LOOP_STAGES = (STAGES // UNROLL) * UNROLL
TAIL = E_TILE - STAGES * CHUNK
WB_TILES = 10
ROWS_PER_TILE = N // WB_TILES


def _pre_body(h_ref, w_ref, out_ref):
    out_ref[...] = jnp.tanh(
        jax.lax.dot_general(h_ref[...], w_ref[0],
                            (((1,), (1,)), ((), ())),
                            preferred_element_type=jnp.float32))


def _make_table(h, W0, W1):
    blk = 2000
    nblk = N // blk
    Ws = jnp.stack([W0, W1])
    return pl.pallas_call(
        _pre_body,
        grid=(2, nblk),
        in_specs=[
            pl.BlockSpec((blk, D), lambda m, i: (i, 0)),
            pl.BlockSpec((1, H, D), lambda m, i: (m, 0, 0)),
        ],
        out_specs=pl.BlockSpec((blk, H), lambda m, i: (m * nblk + i, 0)),
        out_shape=jax.ShapeDtypeStruct((2 * N, H), jnp.float32),
    )(h, Ws)


def _sc_body(table, ei0, ew0, ei1, ew1, binit, e0, e1,
             row_v, col_v, ew_v, msg_v, trow_v, tcol_v, tew_v, acc, *sems):
    sg = sems[0:RING]
    ss = sems[RING:2 * RING]
    sm = sems[2 * RING:3 * RING]
    c = lax.axis_index("c")
    s = lax.axis_index("s")
    ebase = s * E_TILE

    @pl.when(s < WB_TILES)
    def _():
        pltpu.sync_copy(binit.at[c],
                        acc.at[pl.ds(s * ROWS_PER_TILE, ROWS_PER_TILE)])
    plsc.subcore_barrier()

    dnums = lax.GatherDimensionNumbers(
        offset_dims=(), collapsed_slice_dims=(0,), start_index_map=(0,))

    def fire_meta(stage, u):
        base = ebase + stage * CHUNK

        @pl.when(c == 0)
        def _():
            pltpu.async_copy(ei0.at[pl.ds(base, CHUNK)], row_v.at[u], sm[u])
            pltpu.async_copy(ei0.at[pl.ds(E + base, CHUNK)], col_v.at[u],
                             sm[u])
            pltpu.async_copy(ew0.at[pl.ds(base, CHUNK)], ew_v.at[u], sm[u])

        @pl.when(c == 1)
        def _():
            pltpu.async_copy(ei1.at[pl.ds(base, CHUNK)], row_v.at[u], sm[u])
            pltpu.async_copy(ei1.at[pl.ds(E + base, CHUNK)], col_v.at[u],
                             sm[u])
            pltpu.async_copy(ew1.at[pl.ds(base, CHUNK)], ew_v.at[u], sm[u])

    def wait_meta(u):
        pltpu.make_async_copy(ei0.at[pl.ds(0, CHUNK)], row_v.at[u],
                              sm[u]).wait()
        pltpu.make_async_copy(ei0.at[pl.ds(0, CHUNK)], col_v.at[u],
                              sm[u]).wait()
        pltpu.make_async_copy(ew0.at[pl.ds(0, CHUNK)], ew_v.at[u],
                              sm[u]).wait()

        @pl.when(c == 1)
        def _():
            for v in range(CHUNK // 16):
                sl = pl.ds(v * 16, 16)
                col_v[u, sl] = col_v[u, sl] + N

    def fire_gather(u):
        pltpu.async_copy(table.at[col_v.at[u]], msg_v.at[u], sg[u])

    def wait_gather(u):
        pltpu.make_async_copy(table.at[col_v.at[u]], msg_v.at[u],
                              sg[u]).wait()

    def fire_scatter(u):
        pltpu.async_copy(msg_v.at[u], acc.at[row_v.at[u]], ss[u], add=True)

    def wait_scatter(u):
        pltpu.make_async_copy(msg_v.at[u], acc.at[row_v.at[u]],
                              ss[u]).wait()

    def scale_rows(u):
        def grp_body(g, _):
            ew16 = ew_v[u, pl.ds(g * 16, 16)]
            for j in range(16):
                w = lax.gather(ew16, jnp.full((16, 1), j, jnp.int32),
                               dnums, (1,),
                               mode=lax.GatherScatterMode.PROMISE_IN_BOUNDS)
                e = g * 16 + j
                for f in range(H // 16):
                    sl = pl.ds(f * 16, 16)
                    msg_v[u, e, sl] = msg_v[u, e, sl] * w
            return 0

        lax.fori_loop(0, CHUNK // 16, grp_body, 0)

    def stage(k, u):
        @pl.when(k > 0)
        def _():
            wait_scatter((u + RING - 1) % RING)

        @pl.when(k + 4 < STAGES)
        def _():
            fire_meta(k + 4, (u + 4) % RING)

        wait_gather(u)
        scale_rows(u)
        fire_scatter(u)

        @pl.when(k + 3 < STAGES)
        def _():
            wait_meta((u + 3) % RING)
            fire_gather((u + 3) % RING)

    for u in range(4):
        fire_meta(u, u)
    for u in range(3):
        wait_meta(u)
        fire_gather(u)

    def outer_body(i, _):
        for t in range(UNROLL):
            stage(i * UNROLL + t, t % RING)
        return 0

    lax.fori_loop(0, LOOP_STAGES // UNROLL, outer_body, 0)
    for k in range(LOOP_STAGES, STAGES):
        stage(jnp.int32(k), k % RING)
    wait_scatter((STAGES - 1) % RING)

    tbase = ebase + STAGES * CHUNK

    @pl.when(c == 0)
    def _():
        pltpu.sync_copy(ei0.at[pl.ds(tbase, TAIL)], trow_v)
        pltpu.sync_copy(ei0.at[pl.ds(E + tbase, TAIL)], tcol_v)
        pltpu.sync_copy(ew0.at[pl.ds(tbase, TAIL)], tew_v)

    @pl.when(c == 1)
    def _():
        pltpu.sync_copy(ei1.at[pl.ds(tbase, TAIL)], trow_v)
        pltpu.sync_copy(ei1.at[pl.ds(E + tbase, TAIL)], tcol_v)
        pltpu.sync_copy(ew1.at[pl.ds(tbase, TAIL)], tew_v)
        for v in range(TAIL // 16):
            sl = pl.ds(v * 16, 16)
            tcol_v[sl] = tcol_v[sl] + N

    pltpu.async_copy(table.at[tcol_v], msg_v.at[0, pl.ds(0, TAIL)], sg[0])
    pltpu.make_async_copy(table.at[tcol_v],
                          msg_v.at[0, pl.ds(0, TAIL)], sg[0]).wait()

    def tail_grp(g, _):
        ew16 = tew_v[pl.ds(g * 16, 16)]
        for j in range(16):
            w = lax.gather(ew16, jnp.full((16, 1), j, jnp.int32),
                           dnums, (1,),
                           mode=lax.GatherScatterMode.PROMISE_IN_BOUNDS)
            e = g * 16 + j
            for f in range(H // 16):
                sl = pl.ds(f * 16, 16)
                msg_v[0, e, sl] = msg_v[0, e, sl] * w
        return 0

    lax.fori_loop(0, TAIL // 16, tail_grp, 0)
    pltpu.sync_copy(msg_v.at[0, pl.ds(0, TAIL)], acc.at[trow_v], add=True)

    plsc.subcore_barrier()

    sl = pl.ds(s * ROWS_PER_TILE, ROWS_PER_TILE)

    @pl.when(jnp.logical_and(c == 0, s < WB_TILES))
    def _():
        pltpu.sync_copy(acc.at[sl], e0.at[sl])

    @pl.when(jnp.logical_and(c == 1, s < WB_TILES))
    def _():
        pltpu.sync_copy(acc.at[sl], e1.at[sl])


def _segment_spmm(table, ei0, ew0, ei1, ew1, binit):
    mesh = plsc.VectorSubcoreMesh(core_axis_name="c", subcore_axis_name="s")
    f = functools.partial(
        pl.kernel,
        out_type=(jax.ShapeDtypeStruct((N, H), jnp.float32),
                  jax.ShapeDtypeStruct((N, H), jnp.float32)),
        mesh=mesh,
        scratch_types=[
            pltpu.VMEM((RING, CHUNK), jnp.int32),
            pltpu.VMEM((RING, CHUNK), jnp.int32),
            pltpu.VMEM((RING, CHUNK), jnp.float32),
            pltpu.VMEM((RING, CHUNK, H), jnp.float32),
            pltpu.VMEM((TAIL,), jnp.int32),
            pltpu.VMEM((TAIL,), jnp.int32),
            pltpu.VMEM((TAIL,), jnp.float32),
            pltpu.VMEM_SHARED((N, H), jnp.float32),
        ] + [pltpu.SemaphoreType.DMA] * (3 * RING),
    )(_sc_body)
    return f(table, ei0, ew0, ei1, ew1, binit)


def _sums_body(e0_ref, e1_ref, fcw_ref, fcb_ref, out_ref):
    i = pl.program_id(0)
    dn = (((1,), (1,)), ((), ()))
    t0 = jnp.tanh(jax.lax.dot_general(e0_ref[...], fcw_ref[...], dn,
                                      preferred_element_type=jnp.float32)
                  + fcb_ref[...])
    t1 = jnp.tanh(jax.lax.dot_general(e1_ref[...], fcw_ref[...], dn,
                                      preferred_element_type=jnp.float32)
                  + fcb_ref[...])
    part = jnp.stack([jnp.sum(t0, axis=0), jnp.sum(t1, axis=0)])

    @pl.when(i == 0)
    def _():
        out_ref[...] = part

    @pl.when(i > 0)
    def _():
        out_ref[...] = out_ref[...] + part


def _attn_sums(e0, e1, fc_W, fc_b):
    blk = 1000
    nblk = N // blk
    return pl.pallas_call(
        _sums_body,
        grid=(nblk,),
        in_specs=[
            pl.BlockSpec((blk, H), lambda i: (i, 0)),
            pl.BlockSpec((blk, H), lambda i: (i, 0)),
            pl.BlockSpec((H, H), lambda i: (0, 0)),
            pl.BlockSpec((1, H), lambda i: (0, 0)),
        ],
        out_specs=pl.BlockSpec((2, H), lambda i: (0, 0)),
        out_shape=jax.ShapeDtypeStruct((2, H), jnp.float32),
    )(e0, e1, fc_W, fc_b.reshape(1, H))


def _combine_body(sums_ref, att_ref, e0_ref, e1_ref, z_ref):
    sp = sums_ref[...] * (1.0 / N)
    logits = jnp.sum(att_ref[...] * sp, axis=1)
    m = jnp.max(logits)
    ex = jnp.exp(logits - m)
    beta = ex / jnp.sum(ex)
    z_ref[...] = beta[0] * e0_ref[...] + beta[1] * e1_ref[...]


def _combine(sums, att, e0, e1):
    blk = 1000
    nblk = N // blk
    return pl.pallas_call(
        _combine_body,
        grid=(nblk,),
        in_specs=[
            pl.BlockSpec((2, H), lambda i: (0, 0)),
            pl.BlockSpec((1, H), lambda i: (0, 0)),
            pl.BlockSpec((blk, H), lambda i: (i, 0)),
            pl.BlockSpec((blk, H), lambda i: (i, 0)),
        ],
        out_specs=pl.BlockSpec((blk, H), lambda i: (i, 0)),
        out_shape=jax.ShapeDtypeStruct((N, H), jnp.float32),
    )(sums, att, e0, e1)


def kernel(h, mps_edge_index_0, mps_edge_weight_0,
           mps_edge_index_1, mps_edge_weight_1,
           W0, b0, W1, b1, fc_W, fc_b, att):
    ei0 = mps_edge_index_0.astype(jnp.int32).reshape(-1)
    ei1 = mps_edge_index_1.astype(jnp.int32).reshape(-1)
    binit = jnp.stack([
        jnp.broadcast_to(b0[None, :], (ROWS_PER_TILE, H)),
        jnp.broadcast_to(b1[None, :], (ROWS_PER_TILE, H)),
    ])

    table = _make_table(h, W0, W1)
    e0, e1 = _segment_spmm(table, ei0, mps_edge_weight_0,
                           ei1, mps_edge_weight_1, binit)
    sums = _attn_sums(e0, e1, fc_W, fc_b)
    z = _combine(sums, att, e0, e1)
    return (z, e0, e1)
